# baseline (speedup 1.0000x reference)
import functools

import jax
import jax.numpy as jnp
from jax import lax
from jax.experimental import pallas as pl
from jax.experimental.pallas import tpu as pltpu
from jax.experimental.pallas import tpu_sc as plsc

N = 10000
E = 320000
D = 128
DH = D // 2
G = 64
EPS = 1e-10
SCALAR = 20.0

_NC = 2
_NS = 16
_CHUNK = 128
_NBUF = 4
_C = 80
_EPT = _C * _CHUNK
_EPAD = 327680
_RPAD = 10240
_ZR = _RPAD // _NS


def _sc_agg(xt, slo3, shi3, dst3, zeros):
    mesh = plsc.VectorSubcoreMesh(core_axis_name="c", subcore_axis_name="s")

    @functools.partial(
        pl.kernel,
        out_type=jax.ShapeDtypeStruct((_NC, N, DH), jnp.float32),
        mesh=mesh,
        scratch_types=[
            pltpu.VMEM((_C, _CHUNK), jnp.int32),
            pltpu.VMEM((_C, _CHUNK), jnp.int32),
            pltpu.VMEM((_CHUNK, D), jnp.float32),
            pltpu.VMEM((_CHUNK, D), jnp.float32),
            pltpu.VMEM((_CHUNK, D), jnp.float32),
            pltpu.VMEM((_CHUNK, D), jnp.float32),
            pltpu.VMEM_SHARED((_RPAD, DH), jnp.float32),
            pltpu.SemaphoreType.DMA,
            pltpu.SemaphoreType.DMA,
        ],
        compiler_params=pltpu.CompilerParams(use_tc_tiling_on_sc=False),
    )
    def k(xt_hbm, slo_hbm, shi_hbm, dst_hbm, z_hbm, out_hbm,
          sidx, didx, b0, b1, b2, b3, agg, gsem, ssem):
        bufs = (b0, b1, b2, b3)
        c = lax.axis_index("c")
        s = lax.axis_index("s")

        @pl.when(c == 0)
        def _():
            pltpu.sync_copy(slo_hbm.at[s], sidx)

        @pl.when(c == 1)
        def _():
            pltpu.sync_copy(shi_hbm.at[s], sidx)

        pltpu.sync_copy(dst_hbm.at[s], didx)
        pltpu.sync_copy(z_hbm, agg.at[pl.ds(pl.multiple_of(s * _ZR, 8), _ZR)])
        plsc.subcore_barrier()

        for b in range(_NBUF):
            pltpu.make_async_copy(xt_hbm.at[sidx.at[b]], bufs[b], gsem).start()

        def body(jj, carry):
            for b in range(_NBUF):
                j = jj * _NBUF + b
                pltpu.make_async_copy(xt_hbm.at[sidx.at[j]], bufs[b], gsem).wait()

                @pl.when(j + _NBUF < _C)
                def _():
                    pltpu.make_async_copy(
                        xt_hbm.at[sidx.at[j + _NBUF]], bufs[b], gsem).start()
            return carry

        lax.fori_loop(0, _C // _NBUF, body, 0)
        plsc.subcore_barrier()

        @pl.when(s < _NS - 1)
        def _():
            st = pl.multiple_of(s * 624, 8)
            pltpu.sync_copy(agg.at[pl.ds(st, 624)],
                            out_hbm.at[c, pl.ds(st, 624)])

        @pl.when(s == _NS - 1)
        def _():
            pltpu.sync_copy(agg.at[pl.ds(9360, 640)],
                            out_hbm.at[c, pl.ds(9360, 640)])

    return k(xt, slo3, shi3, dst3, zeros)


def _prep(x, node_imp_col, batch_col, batch_row):
    def body(x_ref, imp_ref, bcol_ref, brow_ref, o_ref):
        imp_col = imp_ref[...]
        brow = brow_ref[...]
        gid = lax.broadcasted_iota(jnp.int32, (D, N), 0)
        ohT = (jnp.broadcast_to(brow, (D, N)) == gid)
        imp_row = jnp.broadcast_to(
            jnp.reshape(imp_col, (1, N)), (D, N))
        masked = jnp.where(ohT, imp_row, -3e38)
        segmax_col = jnp.max(masked, axis=1, keepdims=True)
        inv_col = 1.0 / (segmax_col + EPS)
        bcol = bcol_ref[...]
        lane = lax.broadcasted_iota(jnp.int32, (N, D), 1)
        oh = (jnp.broadcast_to(bcol, (N, D)) == lane).astype(jnp.float32)
        inv_node = jnp.dot(oh, inv_col, preferred_element_type=jnp.float32)
        factor = (2.0 * (imp_col * inv_node) - 1.0) / (2.0 * SCALAR) + 1.0
        o_ref[...] = x_ref[...] * factor

    return pl.pallas_call(
        body,
        out_shape=jax.ShapeDtypeStruct((N, D), jnp.float32),
    )(x, node_imp_col, batch_col, batch_row)


def _mlp(x, parts, W1, b1, W2, b2):
    BR = 2000

    def body(x_ref, alo_ref, ahi_ref, w1_ref, b1_ref, w2_ref, b2_ref, o_ref):
        agg = jnp.concatenate([alo_ref[0], ahi_ref[0]], axis=1)
        h = x_ref[...] + agg
        h = jnp.maximum(
            jnp.dot(h, w1_ref[...], preferred_element_type=jnp.float32)
            + b1_ref[...], 0.0)
        o_ref[...] = jnp.maximum(
            jnp.dot(h, w2_ref[...], preferred_element_type=jnp.float32)
            + b2_ref[...], 0.0)

    row = lambda i: (i, 0)
    full = lambda i: (0, 0)
    return pl.pallas_call(
        body,
        grid=(N // BR,),
        in_specs=[
            pl.BlockSpec((BR, D), row),
            pl.BlockSpec((1, BR, DH), lambda i: (0, i, 0)),
            pl.BlockSpec((1, BR, DH), lambda i: (1, i, 0)),
            pl.BlockSpec((D, D), full),
            pl.BlockSpec((1, D), full),
            pl.BlockSpec((D, D), full),
            pl.BlockSpec((1, D), full),
        ],
        out_specs=pl.BlockSpec((BR, D), row),
        out_shape=jax.ShapeDtypeStruct((N, D), jnp.float32),
    )(x, parts, parts, W1, b1, W2, b2)


def _pool(x, batch_row):
    def body(x_ref, brow_ref, o_ref):
        brow = brow_ref[...]
        gid = lax.broadcasted_iota(jnp.int32, (D, N), 0)
        ohT = (jnp.broadcast_to(brow, (D, N)) == gid).astype(jnp.float32)
        sums = jnp.dot(ohT, x_ref[...], preferred_element_type=jnp.float32)
        cnt = jnp.sum(ohT, axis=1, keepdims=True)
        emb = sums / jnp.maximum(cnt, 1.0)
        o_ref[...] = emb[0:G, :]

    return pl.pallas_call(
        body,
        out_shape=jax.ShapeDtypeStruct((G, D), jnp.float32),
    )(x, batch_row)


def kernel(x, edge_index, batch, node_imp,
           W1_0, b1_0, W2_0, b2_0,
           W1_1, b1_1, W2_1, b2_1,
           W1_2, b1_2, W2_2, b2_2):
    src = edge_index[0]
    dst = edge_index[1]
    pad = _EPAD - E
    src_p = jnp.concatenate([src, jnp.zeros((pad,), jnp.int32)])
    dst_p = jnp.concatenate(
        [dst, N + (jnp.arange(pad, dtype=jnp.int32) % (_RPAD - N))])
    half = _EPAD // 2
    slo3 = src_p[:half].reshape(_NS, _C, _CHUNK)
    shi3 = src_p[half:].reshape(_NS, _C, _CHUNK)
    dst3 = dst_p[:half].reshape(_NS, _C, _CHUNK)
    zeros = jnp.zeros((_ZR, DH), jnp.float32)

    batch_col = batch.reshape(N, 1)
    batch_row = batch.reshape(1, N)
    imp_col = node_imp.reshape(N, 1)

    params = [(W1_0, b1_0.reshape(1, D), W2_0, b2_0.reshape(1, D)),
              (W1_1, b1_1.reshape(1, D), W2_1, b2_1.reshape(1, D)),
              (W1_2, b1_2.reshape(1, D), W2_2, b2_2.reshape(1, D))]

    h = _prep(x, imp_col, batch_col, batch_row)
    xs = []
    for (W1, b1, W2, b2) in params:
        parts = _sc_agg(h, slo3, shi3, dst3, zeros)
        h = _mlp(h, parts, W1, b1, W2, b2)
        xs.append(h)
    emb = _pool(h, batch_row)
    return (emb, jnp.concatenate(xs, axis=1))


# speedup vs baseline: 1.2272x; 1.2272x over previous
import functools

import jax
import jax.numpy as jnp
from jax import lax
from jax.experimental import pallas as pl
from jax.experimental.pallas import tpu as pltpu
from jax.experimental.pallas import tpu_sc as plsc

N = 10000
E = 320000
D = 128
DH = D // 2
G = 64
EPS = 1e-10
SCALAR = 20.0

_NC = 2
_NS = 16
_CHUNK = 128
_NBUF = 4
_C = 160
_EPT = _C * _CHUNK
_EPAD = _NS * _EPT
_RPAD = 10240
_ZR = _RPAD // _NS


def _sc_agg(xt, slo3, shi3, dst3):
    mesh = plsc.VectorSubcoreMesh(core_axis_name="c", subcore_axis_name="s")

    @functools.partial(
        pl.kernel,
        out_type=jax.ShapeDtypeStruct((_NC, N, DH), jnp.float32),
        mesh=mesh,
        scratch_types=[
            pltpu.VMEM((_C, _CHUNK), jnp.int32),
            pltpu.VMEM((_C, _CHUNK), jnp.int32),
            pltpu.VMEM((_CHUNK, DH), jnp.float32),
            pltpu.VMEM((_CHUNK, DH), jnp.float32),
            pltpu.VMEM((_CHUNK, DH), jnp.float32),
            pltpu.VMEM((_CHUNK, DH), jnp.float32),
            pltpu.VMEM_SHARED((_RPAD, DH), jnp.float32),
            pltpu.SemaphoreType.DMA,
            pltpu.SemaphoreType.DMA,
        ],
        compiler_params=pltpu.CompilerParams(use_tc_tiling_on_sc=False),
    )
    def k(xt_hbm, slo_hbm, shi_hbm, dst_hbm, z_hbm, out_hbm,
          sidx, didx, b0, b1, b2, b3, agg, gsem, ssem):
        bufs = (b0, b1, b2, b3)
        c = lax.axis_index("c")
        s = lax.axis_index("s")

        @pl.when(c == 0)
        def _():
            pltpu.sync_copy(slo_hbm.at[s], sidx)

        @pl.when(c == 1)
        def _():
            pltpu.sync_copy(shi_hbm.at[s], sidx)

        pltpu.sync_copy(dst_hbm.at[s], didx)

        for b in range(2):
            pltpu.make_async_copy(xt_hbm.at[sidx.at[b]], bufs[b], gsem).start()

        pltpu.make_async_copy(
            z_hbm, agg.at[pl.ds(pl.multiple_of(s * _ZR, 8), _ZR)], ssem).start()
        pltpu.make_async_copy(
            z_hbm, agg.at[pl.ds(pl.multiple_of(s * _ZR, 8), _ZR)], ssem).wait()
        plsc.subcore_barrier()

        def body(jj, carry):
            for b in range(_NBUF):
                j = jj * _NBUF + b
                b4 = (b + 2) % _NBUF
                pltpu.make_async_copy(xt_hbm.at[sidx.at[j]], bufs[b], gsem).wait()
                pltpu.async_copy(bufs[b], agg.at[didx.at[j]], ssem, add=True)

                @pl.when(j + 2 < _C)
                def _():
                    @pl.when(j >= 2)
                    def _():
                        pltpu.make_async_copy(
                            bufs[b4], agg.at[didx.at[j - 2]], ssem).wait()

                    pltpu.make_async_copy(
                        xt_hbm.at[sidx.at[j + 2]], bufs[b4], gsem).start()
            return carry

        lax.fori_loop(0, _C // _NBUF, body, 0)
        for jd in range(_C - 4, _C):
            pltpu.make_async_copy(
                bufs[jd % _NBUF], agg.at[didx.at[jd]], ssem).wait()
        plsc.subcore_barrier()

        @pl.when(s < _NS - 1)
        def _():
            st = pl.multiple_of(s * 624, 8)
            pltpu.sync_copy(agg.at[pl.ds(st, 624)],
                            out_hbm.at[c, pl.ds(st, 624)])

        @pl.when(s == _NS - 1)
        def _():
            pltpu.sync_copy(agg.at[pl.ds(9360, 640)],
                            out_hbm.at[c, pl.ds(9360, 640)])

    return k(xt, slo3, shi3, dst3, jnp.zeros((_ZR, DH), jnp.float32))


def _prep(x, node_imp_col, batch_col, batch_row):
    def body(x_ref, imp_ref, bcol_ref, brow_ref, o_ref):
        imp_col = imp_ref[...]
        brow = brow_ref[...]
        gid = lax.broadcasted_iota(jnp.int32, (D, N), 0)
        ohT = (jnp.broadcast_to(brow, (D, N)) == gid)
        imp_row = jnp.broadcast_to(
            jnp.reshape(imp_col, (1, N)), (D, N))
        masked = jnp.where(ohT, imp_row, -3e38)
        segmax_col = jnp.max(masked, axis=1, keepdims=True)
        inv_col = 1.0 / (segmax_col + EPS)
        bcol = bcol_ref[...]
        lane = lax.broadcasted_iota(jnp.int32, (N, D), 1)
        oh = (jnp.broadcast_to(bcol, (N, D)) == lane).astype(jnp.float32)
        inv_node = jnp.dot(oh, inv_col, preferred_element_type=jnp.float32)
        factor = (2.0 * (imp_col * inv_node) - 1.0) / (2.0 * SCALAR) + 1.0
        o_ref[...] = x_ref[...] * factor

    return pl.pallas_call(
        body,
        out_shape=jax.ShapeDtypeStruct((N, D), jnp.float32),
    )(x, node_imp_col, batch_col, batch_row)


_BR = 2000


def _mlp(x, parts, W1, b1, W2, b2):
    def body(x_ref, alo_ref, ahi_ref, w1_ref, b1_ref, w2_ref, b2_ref, o_ref):
        agg = jnp.concatenate([alo_ref[0], ahi_ref[0]], axis=1)
        h = x_ref[...] + agg
        h = jnp.maximum(
            jnp.dot(h, w1_ref[...], preferred_element_type=jnp.float32)
            + b1_ref[...], 0.0)
        o_ref[...] = jnp.maximum(
            jnp.dot(h, w2_ref[...], preferred_element_type=jnp.float32)
            + b2_ref[...], 0.0)

    row = lambda i: (i, 0)
    full = lambda i: (0, 0)
    return pl.pallas_call(
        body,
        grid=(N // _BR,),
        in_specs=[
            pl.BlockSpec((_BR, D), row),
            pl.BlockSpec((1, _BR, DH), lambda i: (0, i, 0)),
            pl.BlockSpec((1, _BR, DH), lambda i: (1, i, 0)),
            pl.BlockSpec((D, D), full),
            pl.BlockSpec((1, D), full),
            pl.BlockSpec((D, D), full),
            pl.BlockSpec((1, D), full),
        ],
        out_specs=pl.BlockSpec((_BR, D), row),
        out_shape=jax.ShapeDtypeStruct((N, D), jnp.float32),
    )(x, parts, parts, W1, b1, W2, b2)


def _mlp_pool(x, parts, W1, b1, W2, b2, batch_col):
    ngrid = N // _BR

    def body(x_ref, alo_ref, ahi_ref, w1_ref, b1_ref, w2_ref, b2_ref,
             bcol_ref, o_ref, p_ref, acc, cnt):
        i = pl.program_id(0)
        agg = jnp.concatenate([alo_ref[0], ahi_ref[0]], axis=1)
        h = x_ref[...] + agg
        h = jnp.maximum(
            jnp.dot(h, w1_ref[...], preferred_element_type=jnp.float32)
            + b1_ref[...], 0.0)
        xo = jnp.maximum(
            jnp.dot(h, w2_ref[...], preferred_element_type=jnp.float32)
            + b2_ref[...], 0.0)
        o_ref[...] = xo
        bcol = bcol_ref[...]
        lane = lax.broadcasted_iota(jnp.int32, (_BR, D), 1)
        oh = (jnp.broadcast_to(bcol, (_BR, D)) == lane).astype(jnp.float32)
        psum = lax.dot_general(
            oh, xo, (((0,), (0,)), ((), ())),
            preferred_element_type=jnp.float32)
        pcnt = jnp.sum(oh, axis=0, keepdims=True)

        @pl.when(i == 0)
        def _():
            acc[...] = psum
            cnt[...] = pcnt

        @pl.when(i > 0)
        def _():
            acc[...] = acc[...] + psum
            cnt[...] = cnt[...] + pcnt

        @pl.when(i == ngrid - 1)
        def _():
            r = lax.broadcasted_iota(jnp.int32, (D, D), 0)
            l = lax.broadcasted_iota(jnp.int32, (D, D), 1)
            eye = (r == l).astype(jnp.float32)
            cnt_col = lax.dot_general(
                eye, cnt[...], (((1,), (1,)), ((), ())),
                preferred_element_type=jnp.float32)
            p_ref[...] = (acc[...] / jnp.maximum(cnt_col, 1.0))[0:G, :]

    row = lambda i: (i, 0)
    full = lambda i: (0, 0)
    return pl.pallas_call(
        body,
        grid=(ngrid,),
        in_specs=[
            pl.BlockSpec((_BR, D), row),
            pl.BlockSpec((1, _BR, DH), lambda i: (0, i, 0)),
            pl.BlockSpec((1, _BR, DH), lambda i: (1, i, 0)),
            pl.BlockSpec((D, D), full),
            pl.BlockSpec((1, D), full),
            pl.BlockSpec((D, D), full),
            pl.BlockSpec((1, D), full),
            pl.BlockSpec((_BR, 1), lambda i: (i, 0)),
        ],
        out_specs=[
            pl.BlockSpec((_BR, D), row),
            pl.BlockSpec((G, D), full),
        ],
        out_shape=[
            jax.ShapeDtypeStruct((N, D), jnp.float32),
            jax.ShapeDtypeStruct((G, D), jnp.float32),
        ],
        scratch_shapes=[
            pltpu.VMEM((D, D), jnp.float32),
            pltpu.VMEM((1, D), jnp.float32),
        ],
    )(x, parts, parts, W1, b1, W2, b2, batch_col)


def kernel(x, edge_index, batch, node_imp,
           W1_0, b1_0, W2_0, b2_0,
           W1_1, b1_1, W2_1, b2_1,
           W1_2, b1_2, W2_2, b2_2):
    src = edge_index[0]
    dst = edge_index[1]
    pad = _EPAD - E
    src_p = jnp.concatenate([src, jnp.zeros((pad,), jnp.int32)])
    dst_p = jnp.concatenate(
        [dst, N + (jnp.arange(pad, dtype=jnp.int32) % (_RPAD - N))])
    slo3 = (2 * src_p).reshape(_NS, _C, _CHUNK)
    shi3 = (2 * src_p + 1).reshape(_NS, _C, _CHUNK)
    dst3 = dst_p.reshape(_NS, _C, _CHUNK)

    batch_col = batch.reshape(N, 1)
    batch_row = batch.reshape(1, N)
    imp_col = node_imp.reshape(N, 1)

    params = [(W1_0, b1_0.reshape(1, D), W2_0, b2_0.reshape(1, D)),
              (W1_1, b1_1.reshape(1, D), W2_1, b2_1.reshape(1, D)),
              (W1_2, b1_2.reshape(1, D), W2_2, b2_2.reshape(1, D))]

    h = _prep(x, imp_col, batch_col, batch_row)
    xs = []
    for li, (W1, b1, W2, b2) in enumerate(params):
        parts = _sc_agg(h.reshape(2 * N, DH), slo3, shi3, dst3)
        if li < 2:
            h = _mlp(h, parts, W1, b1, W2, b2)
        else:
            h, emb = _mlp_pool(h, parts, W1, b1, W2, b2, batch_col)
        xs.append(h)
    return (emb, jnp.concatenate(xs, axis=1))


# speedup vs baseline: 1.2547x; 1.0224x over previous
import functools

import jax
import jax.numpy as jnp
from jax import lax
from jax.experimental import pallas as pl
from jax.experimental.pallas import tpu as pltpu
from jax.experimental.pallas import tpu_sc as plsc

N = 10000
E = 320000
D = 128
DH = D // 2
G = 64
EPS = 1e-10
SCALAR = 20.0

_NC = 2
_NS = 16
_CHUNK = 128
_NBUF = 4
_C = 160
_EPT = _C * _CHUNK
_EPAD = _NS * _EPT
_RPAD = 10240
_ZR = _RPAD // _NS


def _sc_agg(xt, slo3, shi3, dst3):
    mesh = plsc.VectorSubcoreMesh(core_axis_name="c", subcore_axis_name="s")

    @functools.partial(
        pl.kernel,
        out_type=jax.ShapeDtypeStruct((_NC, N, DH), jnp.float32),
        mesh=mesh,
        scratch_types=[
            pltpu.VMEM((_C, _CHUNK), jnp.int32),
            pltpu.VMEM((_C, _CHUNK), jnp.int32),
            pltpu.VMEM((_CHUNK, DH), jnp.float32),
            pltpu.VMEM((_CHUNK, DH), jnp.float32),
            pltpu.VMEM((_CHUNK, DH), jnp.float32),
            pltpu.VMEM((_CHUNK, DH), jnp.float32),
            pltpu.VMEM_SHARED((_RPAD, DH), jnp.float32),
            pltpu.SemaphoreType.DMA,
            pltpu.SemaphoreType.DMA,
        ],
        compiler_params=pltpu.CompilerParams(use_tc_tiling_on_sc=False),
    )
    def k(xt_hbm, slo_hbm, shi_hbm, dst_hbm, z_hbm, out_hbm,
          sidx, didx, b0, b1, b2, b3, agg, gsem, ssem):
        bufs = (b0, b1, b2, b3)
        c = lax.axis_index("c")
        s = lax.axis_index("s")

        @pl.when(c == 0)
        def _():
            pltpu.sync_copy(slo_hbm.at[s], sidx)

        @pl.when(c == 1)
        def _():
            pltpu.sync_copy(shi_hbm.at[s], sidx)

        pltpu.sync_copy(dst_hbm.at[s], didx)

        for b in range(_NBUF):
            pltpu.make_async_copy(xt_hbm.at[sidx.at[b]], bufs[b], gsem).start()

        pltpu.make_async_copy(
            z_hbm, agg.at[pl.ds(pl.multiple_of(s * _ZR, 8), _ZR)], ssem).start()
        pltpu.make_async_copy(
            z_hbm, agg.at[pl.ds(pl.multiple_of(s * _ZR, 8), _ZR)], ssem).wait()
        plsc.subcore_barrier()

        def body(jj, carry):
            for b in range(_NBUF):
                j = jj * _NBUF + b
                pltpu.make_async_copy(xt_hbm.at[sidx.at[j]], bufs[b], gsem).wait()
                pltpu.sync_copy(bufs[b], agg.at[didx.at[j]], add=True)

                @pl.when(j + _NBUF < _C)
                def _():
                    pltpu.make_async_copy(
                        xt_hbm.at[sidx.at[j + _NBUF]], bufs[b], gsem).start()
            return carry

        lax.fori_loop(0, _C // _NBUF, body, 0)
        plsc.subcore_barrier()

        @pl.when(s < _NS - 1)
        def _():
            st = pl.multiple_of(s * 624, 8)
            pltpu.sync_copy(agg.at[pl.ds(st, 624)],
                            out_hbm.at[c, pl.ds(st, 624)])

        @pl.when(s == _NS - 1)
        def _():
            pltpu.sync_copy(agg.at[pl.ds(9360, 640)],
                            out_hbm.at[c, pl.ds(9360, 640)])

    return k(xt, slo3, shi3, dst3, jnp.zeros((_ZR, DH), jnp.float32))


def _prep(x, node_imp_col, batch_col, batch_row):
    def body(x_ref, imp_ref, bcol_ref, brow_ref, o_ref):
        imp_col = imp_ref[...]
        brow = brow_ref[...]
        gid = lax.broadcasted_iota(jnp.int32, (D, N), 0)
        ohT = (jnp.broadcast_to(brow, (D, N)) == gid)
        imp_row = jnp.broadcast_to(
            jnp.reshape(imp_col, (1, N)), (D, N))
        masked = jnp.where(ohT, imp_row, -3e38)
        segmax_col = jnp.max(masked, axis=1, keepdims=True)
        inv_col = 1.0 / (segmax_col + EPS)
        bcol = bcol_ref[...]
        lane = lax.broadcasted_iota(jnp.int32, (N, D), 1)
        oh = (jnp.broadcast_to(bcol, (N, D)) == lane).astype(jnp.float32)
        inv_node = jnp.dot(oh, inv_col, preferred_element_type=jnp.float32)
        factor = (2.0 * (imp_col * inv_node) - 1.0) / (2.0 * SCALAR) + 1.0
        o_ref[...] = x_ref[...] * factor

    return pl.pallas_call(
        body,
        out_shape=jax.ShapeDtypeStruct((N, D), jnp.float32),
    )(x, node_imp_col, batch_col, batch_row)


_BR = 2000


def _mlp(x, parts, W1, b1, W2, b2):
    def body(x_ref, alo_ref, ahi_ref, w1_ref, b1_ref, w2_ref, b2_ref, o_ref):
        agg = jnp.concatenate([alo_ref[0], ahi_ref[0]], axis=1)
        h = x_ref[...] + agg
        h = jnp.maximum(
            jnp.dot(h, w1_ref[...], preferred_element_type=jnp.float32)
            + b1_ref[...], 0.0)
        o_ref[...] = jnp.maximum(
            jnp.dot(h, w2_ref[...], preferred_element_type=jnp.float32)
            + b2_ref[...], 0.0)

    row = lambda i: (i, 0)
    full = lambda i: (0, 0)
    return pl.pallas_call(
        body,
        grid=(N // _BR,),
        in_specs=[
            pl.BlockSpec((_BR, D), row),
            pl.BlockSpec((1, _BR, DH), lambda i: (0, i, 0)),
            pl.BlockSpec((1, _BR, DH), lambda i: (1, i, 0)),
            pl.BlockSpec((D, D), full),
            pl.BlockSpec((1, D), full),
            pl.BlockSpec((D, D), full),
            pl.BlockSpec((1, D), full),
        ],
        out_specs=pl.BlockSpec((_BR, D), row),
        out_shape=jax.ShapeDtypeStruct((N, D), jnp.float32),
    )(x, parts, parts, W1, b1, W2, b2)


def _mlp_pool(x, parts, W1, b1, W2, b2, batch_col):
    ngrid = N // _BR

    def body(x_ref, alo_ref, ahi_ref, w1_ref, b1_ref, w2_ref, b2_ref,
             bcol_ref, o_ref, p_ref, acc, cnt):
        i = pl.program_id(0)
        agg = jnp.concatenate([alo_ref[0], ahi_ref[0]], axis=1)
        h = x_ref[...] + agg
        h = jnp.maximum(
            jnp.dot(h, w1_ref[...], preferred_element_type=jnp.float32)
            + b1_ref[...], 0.0)
        xo = jnp.maximum(
            jnp.dot(h, w2_ref[...], preferred_element_type=jnp.float32)
            + b2_ref[...], 0.0)
        o_ref[...] = xo
        bcol = bcol_ref[...]
        lane = lax.broadcasted_iota(jnp.int32, (_BR, D), 1)
        oh = (jnp.broadcast_to(bcol, (_BR, D)) == lane).astype(jnp.float32)
        psum = lax.dot_general(
            oh, xo, (((0,), (0,)), ((), ())),
            preferred_element_type=jnp.float32)
        pcnt = jnp.sum(oh, axis=0, keepdims=True)

        @pl.when(i == 0)
        def _():
            acc[...] = psum
            cnt[...] = pcnt

        @pl.when(i > 0)
        def _():
            acc[...] = acc[...] + psum
            cnt[...] = cnt[...] + pcnt

        @pl.when(i == ngrid - 1)
        def _():
            r = lax.broadcasted_iota(jnp.int32, (D, D), 0)
            l = lax.broadcasted_iota(jnp.int32, (D, D), 1)
            eye = (r == l).astype(jnp.float32)
            cnt_col = lax.dot_general(
                eye, cnt[...], (((1,), (1,)), ((), ())),
                preferred_element_type=jnp.float32)
            p_ref[...] = (acc[...] / jnp.maximum(cnt_col, 1.0))[0:G, :]

    row = lambda i: (i, 0)
    full = lambda i: (0, 0)
    return pl.pallas_call(
        body,
        grid=(ngrid,),
        in_specs=[
            pl.BlockSpec((_BR, D), row),
            pl.BlockSpec((1, _BR, DH), lambda i: (0, i, 0)),
            pl.BlockSpec((1, _BR, DH), lambda i: (1, i, 0)),
            pl.BlockSpec((D, D), full),
            pl.BlockSpec((1, D), full),
            pl.BlockSpec((D, D), full),
            pl.BlockSpec((1, D), full),
            pl.BlockSpec((_BR, 1), lambda i: (i, 0)),
        ],
        out_specs=[
            pl.BlockSpec((_BR, D), row),
            pl.BlockSpec((G, D), full),
        ],
        out_shape=[
            jax.ShapeDtypeStruct((N, D), jnp.float32),
            jax.ShapeDtypeStruct((G, D), jnp.float32),
        ],
        scratch_shapes=[
            pltpu.VMEM((D, D), jnp.float32),
            pltpu.VMEM((1, D), jnp.float32),
        ],
    )(x, parts, parts, W1, b1, W2, b2, batch_col)


def kernel(x, edge_index, batch, node_imp,
           W1_0, b1_0, W2_0, b2_0,
           W1_1, b1_1, W2_1, b2_1,
           W1_2, b1_2, W2_2, b2_2):
    src = edge_index[0]
    dst = edge_index[1]
    pad = _EPAD - E
    src_p = jnp.concatenate([src, jnp.zeros((pad,), jnp.int32)])
    dst_p = jnp.concatenate(
        [dst, N + (jnp.arange(pad, dtype=jnp.int32) % (_RPAD - N))])
    slo3 = (2 * src_p).reshape(_NS, _C, _CHUNK)
    shi3 = (2 * src_p + 1).reshape(_NS, _C, _CHUNK)
    dst3 = dst_p.reshape(_NS, _C, _CHUNK)

    batch_col = batch.reshape(N, 1)
    batch_row = batch.reshape(1, N)
    imp_col = node_imp.reshape(N, 1)

    params = [(W1_0, b1_0.reshape(1, D), W2_0, b2_0.reshape(1, D)),
              (W1_1, b1_1.reshape(1, D), W2_1, b2_1.reshape(1, D)),
              (W1_2, b1_2.reshape(1, D), W2_2, b2_2.reshape(1, D))]

    h = _prep(x, imp_col, batch_col, batch_row)
    xs = []
    for li, (W1, b1, W2, b2) in enumerate(params):
        parts = _sc_agg(h.reshape(2 * N, DH), slo3, shi3, dst3)
        if li < 2:
            h = _mlp(h, parts, W1, b1, W2, b2)
        else:
            h, emb = _mlp_pool(h, parts, W1, b1, W2, b2, batch_col)
        xs.append(h)
    return (emb, jnp.concatenate(xs, axis=1))


# speedup vs baseline: 1.2563x; 1.0013x over previous
import functools

import jax
import jax.numpy as jnp
from jax import lax
from jax.experimental import pallas as pl
from jax.experimental.pallas import tpu as pltpu
from jax.experimental.pallas import tpu_sc as plsc

N = 10000
E = 320000
D = 128
DH = D // 2
G = 64
EPS = 1e-10
SCALAR = 20.0

_NC = 2
_NS = 16
_CHUNK = 128
_NBUF = 5
_C = 160
_EPT = _C * _CHUNK
_EPAD = _NS * _EPT
_RPAD = 10240
_ZR = _RPAD // _NS


def _sc_agg(xt, slo3, shi3, dst3):
    mesh = plsc.VectorSubcoreMesh(core_axis_name="c", subcore_axis_name="s")

    @functools.partial(
        pl.kernel,
        out_type=jax.ShapeDtypeStruct((_NC, N, DH), jnp.float32),
        mesh=mesh,
        scratch_types=[
            pltpu.VMEM((_C, _CHUNK), jnp.int32),
            pltpu.VMEM((_C, _CHUNK), jnp.int32),
            pltpu.VMEM((_CHUNK, DH), jnp.float32),
            pltpu.VMEM((_CHUNK, DH), jnp.float32),
            pltpu.VMEM((_CHUNK, DH), jnp.float32),
            pltpu.VMEM((_CHUNK, DH), jnp.float32),
            pltpu.VMEM((_CHUNK, DH), jnp.float32),
            pltpu.VMEM_SHARED((_RPAD, DH), jnp.float32),
            pltpu.SemaphoreType.DMA,
            pltpu.SemaphoreType.DMA,
        ],
        compiler_params=pltpu.CompilerParams(use_tc_tiling_on_sc=False),
    )
    def k(xt_hbm, slo_hbm, shi_hbm, dst_hbm, z_hbm, out_hbm,
          sidx, didx, b0, b1, b2, b3, b4, agg, gsem, ssem):
        bufs = (b0, b1, b2, b3, b4)
        c = lax.axis_index("c")
        s = lax.axis_index("s")

        @pl.when(c == 0)
        def _():
            pltpu.sync_copy(slo_hbm.at[s], sidx)

        @pl.when(c == 1)
        def _():
            pltpu.sync_copy(shi_hbm.at[s], sidx)

        pltpu.sync_copy(dst_hbm.at[s], didx)

        for b in range(_NBUF):
            pltpu.make_async_copy(xt_hbm.at[sidx.at[b]], bufs[b], gsem).start()

        pltpu.make_async_copy(
            z_hbm, agg.at[pl.ds(pl.multiple_of(s * _ZR, 8), _ZR)], ssem).start()
        pltpu.make_async_copy(
            z_hbm, agg.at[pl.ds(pl.multiple_of(s * _ZR, 8), _ZR)], ssem).wait()
        plsc.subcore_barrier()

        def body(jj, carry):
            for b in range(_NBUF):
                j = jj * _NBUF + b
                pltpu.make_async_copy(xt_hbm.at[sidx.at[j]], bufs[b], gsem).wait()
                pltpu.sync_copy(bufs[b], agg.at[didx.at[j]], add=True)

                @pl.when(j + _NBUF < _C)
                def _():
                    pltpu.make_async_copy(
                        xt_hbm.at[sidx.at[j + _NBUF]], bufs[b], gsem).start()
            return carry

        lax.fori_loop(0, _C // _NBUF, body, 0)
        plsc.subcore_barrier()

        @pl.when(s < _NS - 1)
        def _():
            st = pl.multiple_of(s * 624, 8)
            pltpu.sync_copy(agg.at[pl.ds(st, 624)],
                            out_hbm.at[c, pl.ds(st, 624)])

        @pl.when(s == _NS - 1)
        def _():
            pltpu.sync_copy(agg.at[pl.ds(9360, 640)],
                            out_hbm.at[c, pl.ds(9360, 640)])

    return k(xt, slo3, shi3, dst3, jnp.zeros((_ZR, DH), jnp.float32))


def _prep(x, node_imp_col, batch_col, batch_row):
    def body(x_ref, imp_ref, bcol_ref, brow_ref, o_ref):
        imp_col = imp_ref[...]
        brow = brow_ref[...]
        gid = lax.broadcasted_iota(jnp.int32, (D, N), 0)
        ohT = (jnp.broadcast_to(brow, (D, N)) == gid)
        imp_row = jnp.broadcast_to(
            jnp.reshape(imp_col, (1, N)), (D, N))
        masked = jnp.where(ohT, imp_row, -3e38)
        segmax_col = jnp.max(masked, axis=1, keepdims=True)
        inv_col = 1.0 / (segmax_col + EPS)
        bcol = bcol_ref[...]
        lane = lax.broadcasted_iota(jnp.int32, (N, D), 1)
        oh = (jnp.broadcast_to(bcol, (N, D)) == lane).astype(jnp.float32)
        inv_node = jnp.dot(oh, inv_col, preferred_element_type=jnp.float32)
        factor = (2.0 * (imp_col * inv_node) - 1.0) / (2.0 * SCALAR) + 1.0
        o_ref[...] = x_ref[...] * factor

    return pl.pallas_call(
        body,
        out_shape=jax.ShapeDtypeStruct((N, D), jnp.float32),
    )(x, node_imp_col, batch_col, batch_row)


_BR = 2000


def _mlp(x, parts, W1, b1, W2, b2):
    def body(x_ref, alo_ref, ahi_ref, w1_ref, b1_ref, w2_ref, b2_ref, o_ref):
        agg = jnp.concatenate([alo_ref[0], ahi_ref[0]], axis=1)
        h = x_ref[...] + agg
        h = jnp.maximum(
            jnp.dot(h, w1_ref[...], preferred_element_type=jnp.float32)
            + b1_ref[...], 0.0)
        o_ref[...] = jnp.maximum(
            jnp.dot(h, w2_ref[...], preferred_element_type=jnp.float32)
            + b2_ref[...], 0.0)

    row = lambda i: (i, 0)
    full = lambda i: (0, 0)
    return pl.pallas_call(
        body,
        grid=(N // _BR,),
        in_specs=[
            pl.BlockSpec((_BR, D), row),
            pl.BlockSpec((1, _BR, DH), lambda i: (0, i, 0)),
            pl.BlockSpec((1, _BR, DH), lambda i: (1, i, 0)),
            pl.BlockSpec((D, D), full),
            pl.BlockSpec((1, D), full),
            pl.BlockSpec((D, D), full),
            pl.BlockSpec((1, D), full),
        ],
        out_specs=pl.BlockSpec((_BR, D), row),
        out_shape=jax.ShapeDtypeStruct((N, D), jnp.float32),
    )(x, parts, parts, W1, b1, W2, b2)


def _mlp_pool(x, parts, W1, b1, W2, b2, batch_col):
    ngrid = N // _BR

    def body(x_ref, alo_ref, ahi_ref, w1_ref, b1_ref, w2_ref, b2_ref,
             bcol_ref, o_ref, p_ref, acc, cnt):
        i = pl.program_id(0)
        agg = jnp.concatenate([alo_ref[0], ahi_ref[0]], axis=1)
        h = x_ref[...] + agg
        h = jnp.maximum(
            jnp.dot(h, w1_ref[...], preferred_element_type=jnp.float32)
            + b1_ref[...], 0.0)
        xo = jnp.maximum(
            jnp.dot(h, w2_ref[...], preferred_element_type=jnp.float32)
            + b2_ref[...], 0.0)
        o_ref[...] = xo
        bcol = bcol_ref[...]
        lane = lax.broadcasted_iota(jnp.int32, (_BR, D), 1)
        oh = (jnp.broadcast_to(bcol, (_BR, D)) == lane).astype(jnp.float32)
        psum = lax.dot_general(
            oh, xo, (((0,), (0,)), ((), ())),
            preferred_element_type=jnp.float32)
        pcnt = jnp.sum(oh, axis=0, keepdims=True)

        @pl.when(i == 0)
        def _():
            acc[...] = psum
            cnt[...] = pcnt

        @pl.when(i > 0)
        def _():
            acc[...] = acc[...] + psum
            cnt[...] = cnt[...] + pcnt

        @pl.when(i == ngrid - 1)
        def _():
            r = lax.broadcasted_iota(jnp.int32, (D, D), 0)
            l = lax.broadcasted_iota(jnp.int32, (D, D), 1)
            eye = (r == l).astype(jnp.float32)
            cnt_col = lax.dot_general(
                eye, cnt[...], (((1,), (1,)), ((), ())),
                preferred_element_type=jnp.float32)
            p_ref[...] = (acc[...] / jnp.maximum(cnt_col, 1.0))[0:G, :]

    row = lambda i: (i, 0)
    full = lambda i: (0, 0)
    return pl.pallas_call(
        body,
        grid=(ngrid,),
        in_specs=[
            pl.BlockSpec((_BR, D), row),
            pl.BlockSpec((1, _BR, DH), lambda i: (0, i, 0)),
            pl.BlockSpec((1, _BR, DH), lambda i: (1, i, 0)),
            pl.BlockSpec((D, D), full),
            pl.BlockSpec((1, D), full),
            pl.BlockSpec((D, D), full),
            pl.BlockSpec((1, D), full),
            pl.BlockSpec((_BR, 1), lambda i: (i, 0)),
        ],
        out_specs=[
            pl.BlockSpec((_BR, D), row),
            pl.BlockSpec((G, D), full),
        ],
        out_shape=[
            jax.ShapeDtypeStruct((N, D), jnp.float32),
            jax.ShapeDtypeStruct((G, D), jnp.float32),
        ],
        scratch_shapes=[
            pltpu.VMEM((D, D), jnp.float32),
            pltpu.VMEM((1, D), jnp.float32),
        ],
    )(x, parts, parts, W1, b1, W2, b2, batch_col)


def kernel(x, edge_index, batch, node_imp,
           W1_0, b1_0, W2_0, b2_0,
           W1_1, b1_1, W2_1, b2_1,
           W1_2, b1_2, W2_2, b2_2):
    src = edge_index[0]
    dst = edge_index[1]
    pad = _EPAD - E
    src_p = jnp.concatenate([src, jnp.zeros((pad,), jnp.int32)])
    dst_p = jnp.concatenate(
        [dst, N + (jnp.arange(pad, dtype=jnp.int32) % (_RPAD - N))])
    slo3 = (2 * src_p).reshape(_NS, _C, _CHUNK)
    shi3 = (2 * src_p + 1).reshape(_NS, _C, _CHUNK)
    dst3 = dst_p.reshape(_NS, _C, _CHUNK)

    batch_col = batch.reshape(N, 1)
    batch_row = batch.reshape(1, N)
    imp_col = node_imp.reshape(N, 1)

    params = [(W1_0, b1_0.reshape(1, D), W2_0, b2_0.reshape(1, D)),
              (W1_1, b1_1.reshape(1, D), W2_1, b2_1.reshape(1, D)),
              (W1_2, b1_2.reshape(1, D), W2_2, b2_2.reshape(1, D))]

    h = _prep(x, imp_col, batch_col, batch_row)
    xs = []
    for li, (W1, b1, W2, b2) in enumerate(params):
        parts = _sc_agg(h.reshape(2 * N, DH), slo3, shi3, dst3)
        if li < 2:
            h = _mlp(h, parts, W1, b1, W2, b2)
        else:
            h, emb = _mlp_pool(h, parts, W1, b1, W2, b2, batch_col)
        xs.append(h)
    return (emb, jnp.concatenate(xs, axis=1))
